# baseline (speedup 1.0000x reference)
import jax
import jax.numpy as jnp
from jax import lax
from jax.experimental import pallas as pl
from jax.experimental.pallas import tpu as pltpu
from jax.experimental.pallas import tpu_sc as plsc

NC = 2
NS = 16
NW = NC * NS

BATCH = 16384
HIST = 50
EMBED_DIM = 32
B_PER_W = BATCH // NW
HP = 56
LB = 128
NBL = B_PER_W // LB
NU = HIST * NBL
SUBL = 8
NDT = EMBED_DIM // SUBL


def _gather_body(idx_hbm, table_hbm, out_hbm, idx_v, idx_t, rows_a, rows_b,
                 t_a, t_b, gs_a, gs_b, ws_a, ws_b):
    wid = lax.axis_index("s") * NC + lax.axis_index("c")
    base = wid * B_PER_W
    pltpu.sync_copy(idx_hbm.at[pl.ds(base * HP, B_PER_W * HP)], idx_v)

    iota = lax.iota(jnp.int32, 16)

    @pl.loop(0, HIST)
    def _idx_t(h):
        for c in range(B_PER_W // 16):
            src = HP * (16 * c + iota) + h
            val = plsc.load_gather(idx_v, [src])
            idx_t[pl.ds(h * B_PER_W + 16 * c, 16)] = val

    rows = (rows_a, rows_b)
    tt = (t_a, t_b)
    gsem = (gs_a, gs_b)
    wsem = (ws_a, ws_b)

    def fire_gather(u, buf):
        h = u // NBL
        bl = lax.rem(u, NBL)
        pltpu.async_copy(
            table_hbm.at[idx_t.at[pl.ds(h * B_PER_W + bl * LB, LB)]],
            rows[buf], gsem[buf])

    def drain_gather(buf):
        pltpu.make_async_copy(
            table_hbm.at[pl.ds(0, LB)], rows[buf], gsem[buf]).wait()

    def transpose_unit(buf):
        for d in range(EMBED_DIM):
            col = jnp.full((16,), d, jnp.int32)
            for c in range(LB // 16):
                val = plsc.load_gather(rows[buf], [16 * c + iota, col])
                tt[buf][pl.ds(d * LB + 16 * c, 16)] = val

    def write_unit(u, buf):
        h = u // NBL
        bl = lax.rem(u, NBL)
        for dt in range(NDT):
            pltpu.async_copy(
                tt[buf].at[pl.ds(dt * SUBL * LB, SUBL * LB)],
                out_hbm.at[h, dt, wid * NBL + bl], wsem[buf])

    def wait_write(u, buf):
        h = u // NBL
        bl = lax.rem(u, NBL)
        for dt in range(NDT):
            pltpu.make_async_copy(
                tt[buf].at[pl.ds(dt * SUBL * LB, SUBL * LB)],
                out_hbm.at[h, dt, wid * NBL + bl], wsem[buf]).wait()

    fire_gather(0, 0)

    @pl.loop(0, NU, step=2)
    def _outer(u0):
        for b in (0, 1):
            u = u0 + b
            nxt = 1 - b

            @pl.when(u + 1 < NU)
            def _fire():
                @pl.when(u >= 1)
                def _w():
                    wait_write(u - 1, nxt)
                fire_gather(u + 1, nxt)

            drain_gather(b)
            transpose_unit(b)
            write_unit(u, b)

    wait_write(NU - 2, 0)
    wait_write(NU - 1, 1)


def kernel(inputs, table):
    idx = jnp.pad(inputs.astype(jnp.int32), ((0, 0), (0, HP - HIST))).reshape(-1)
    mesh = plsc.VectorSubcoreMesh(
        core_axis_name="c", subcore_axis_name="s", num_cores=NC, num_subcores=NS
    )
    out5 = pl.kernel(
        _gather_body,
        out_type=jax.ShapeDtypeStruct(
            (HIST, NDT, BATCH // LB, SUBL * LB), jnp.float32),
        mesh=mesh,
        compiler_params=pltpu.CompilerParams(
            use_tc_tiling_on_sc=False, needs_layout_passes=False),
        scratch_types=[
            pltpu.VMEM((B_PER_W * HP,), jnp.int32),
            pltpu.VMEM((HIST * B_PER_W,), jnp.int32),
            pltpu.VMEM((LB, EMBED_DIM), jnp.float32),
            pltpu.VMEM((LB, EMBED_DIM), jnp.float32),
            pltpu.VMEM((EMBED_DIM * LB,), jnp.float32),
            pltpu.VMEM((EMBED_DIM * LB,), jnp.float32),
            pltpu.SemaphoreType.DMA,
            pltpu.SemaphoreType.DMA,
            pltpu.SemaphoreType.DMA,
            pltpu.SemaphoreType.DMA,
        ],
    )(idx, table)
    return (out5.reshape(HIST, NDT, BATCH // LB, SUBL, LB)
            .transpose(2, 4, 0, 1, 3)
            .reshape(BATCH, HIST, EMBED_DIM))


# speedup vs baseline: 1.6517x; 1.6517x over previous
import jax
import jax.numpy as jnp
from jax import lax
from jax.experimental import pallas as pl
from jax.experimental.pallas import tpu as pltpu
from jax.experimental.pallas import tpu_sc as plsc

NC = 2
NS = 16
NW = NC * NS

BATCH = 16384
HIST = 50
EMBED_DIM = 32
B_PER_W = BATCH // NW
HP = 56
LB = 128
NBL = B_PER_W // LB
NU = HIST * NBL
SUBL = 8
NDT = EMBED_DIM // SUBL


def _gather_body(idx_hbm, table_hbm, out_hbm, idx_v, idx_t, rows_a, rows_b,
                 t_a, t_b, gs_a, gs_b, ws_a, ws_b):
    wid = lax.axis_index("s") * NC + lax.axis_index("c")
    base = wid * B_PER_W
    pltpu.sync_copy(idx_hbm.at[pl.ds(base * HP, B_PER_W * HP)], idx_v)

    iota = lax.iota(jnp.int32, 16)

    @pl.loop(0, HIST)
    def _idx_t(h):
        for c in range(B_PER_W // 16):
            src = HP * (16 * c + iota) + h
            val = plsc.load_gather(idx_v, [src])
            idx_t[pl.ds(h * B_PER_W + 16 * c, 16)] = val

    rows = (rows_a, rows_b)
    tt = (t_a, t_b)
    gsem = (gs_a, gs_b)
    wsem = (ws_a, ws_b)

    def fire_gather(u, buf):
        h = u // NBL
        bl = lax.rem(u, NBL)
        pltpu.async_copy(
            table_hbm.at[idx_t.at[pl.ds(h * B_PER_W + bl * LB, LB)]],
            rows[buf], gsem[buf])

    def drain_gather(buf):
        pltpu.make_async_copy(
            table_hbm.at[pl.ds(0, LB)], rows[buf], gsem[buf]).wait()

    def transpose_unit(buf):
        for l in range(LB):
            for k in range(EMBED_DIM // 16):
                val = rows[buf][l, pl.ds(16 * k, 16)]
                plsc.store_scatter(
                    tt[buf], [16 * k + iota, jnp.full((16,), l, jnp.int32)],
                    val)

    def write_unit(u, buf):
        h = u // NBL
        bl = lax.rem(u, NBL)
        for dt in range(NDT):
            pltpu.async_copy(
                tt[buf].at[pl.ds(dt * SUBL, SUBL), pl.ds(0, LB)],
                out_hbm.at[h, dt, wid * NBL + bl], wsem[buf])

    def wait_write(u, buf):
        h = u // NBL
        bl = lax.rem(u, NBL)
        for dt in range(NDT):
            pltpu.make_async_copy(
                tt[buf].at[pl.ds(dt * SUBL, SUBL), pl.ds(0, LB)],
                out_hbm.at[h, dt, wid * NBL + bl], wsem[buf]).wait()

    fire_gather(0, 0)

    @pl.loop(0, NU, step=2)
    def _outer(u0):
        for b in (0, 1):
            u = u0 + b
            nxt = 1 - b

            @pl.when(u + 1 < NU)
            def _fire():
                @pl.when(u >= 1)
                def _w():
                    wait_write(u - 1, nxt)
                fire_gather(u + 1, nxt)

            drain_gather(b)
            transpose_unit(b)
            write_unit(u, b)

    wait_write(NU - 2, 0)
    wait_write(NU - 1, 1)


def kernel(inputs, table):
    idx = jnp.pad(inputs.astype(jnp.int32), ((0, 0), (0, HP - HIST))).reshape(-1)
    mesh = plsc.VectorSubcoreMesh(
        core_axis_name="c", subcore_axis_name="s", num_cores=NC, num_subcores=NS
    )
    out5 = pl.kernel(
        _gather_body,
        out_type=jax.ShapeDtypeStruct(
            (HIST, NDT, BATCH // LB, SUBL, LB), jnp.float32),
        mesh=mesh,
        compiler_params=pltpu.CompilerParams(
            use_tc_tiling_on_sc=False, needs_layout_passes=False),
        scratch_types=[
            pltpu.VMEM((B_PER_W * HP,), jnp.int32),
            pltpu.VMEM((HIST * B_PER_W,), jnp.int32),
            pltpu.VMEM((LB, EMBED_DIM), jnp.float32),
            pltpu.VMEM((LB, EMBED_DIM), jnp.float32),
            pltpu.VMEM((EMBED_DIM, LB + 1), jnp.float32),
            pltpu.VMEM((EMBED_DIM, LB + 1), jnp.float32),
            pltpu.SemaphoreType.DMA,
            pltpu.SemaphoreType.DMA,
            pltpu.SemaphoreType.DMA,
            pltpu.SemaphoreType.DMA,
        ],
    )(idx, table)
    return out5.transpose(2, 4, 0, 1, 3).reshape(BATCH, HIST, EMBED_DIM)
